# Initial kernel scaffold; baseline (speedup 1.0000x reference)
import jax
import jax.numpy as jnp
from jax.experimental import pallas as pl


def kernel(predictions, relevance_scores):
    raise NotImplementedError("write your pallas kernel here")


# speedup vs baseline: 1.3962x; 1.3962x over previous
import functools

import numpy as np
import jax
import jax.numpy as jnp
from jax import lax
from jax.experimental import pallas as pl
from jax.experimental.pallas import tpu as pltpu
from jax.experimental.pallas import tpu_sc as plsc

B = 1024
N = 100000
K = 10
NC = 2
NS = 16
NW = NC * NS
ROWS_PW = B // NW
CHUNK = 20000
GROUP = 160
NVREG = GROUP // 16
NGROUPS = CHUNK // GROUP
NCHUNKS = N // CHUNK

_W = np.zeros(16, np.float32)
_W[:K] = (1.0 / np.log2(np.arange(1, K + 1, dtype=np.float64) + 1.0)).astype(
    np.float32)


def _ndcg_rows(pred_flat, rel_flat):
  mesh = plsc.VectorSubcoreMesh(
      core_axis_name="c", subcore_axis_name="s", num_cores=NC,
      num_subcores=NS)

  @functools.partial(
      pl.kernel,
      out_type=jax.ShapeDtypeStruct((B,), jnp.float32),
      mesh=mesh,
      compiler_params=pltpu.CompilerParams(needs_layout_passes=False),
      scratch_types=[
          pltpu.VMEM((CHUNK,), jnp.float32),
          pltpu.VMEM((CHUNK,), jnp.float32),
          pltpu.VMEM((16,), jnp.float32),
          pltpu.VMEM((16,), jnp.int32),
          pltpu.VMEM((16,), jnp.float32),
          pltpu.VMEM((16,), jnp.float32),
          pltpu.VMEM((16,), jnp.float32),
          pltpu.VMEM((16,), jnp.float32),
          pltpu.VMEM((ROWS_PW,), jnp.float32),
          pltpu.SemaphoreType.DMA,
      ],
  )
  def ndcg_kernel(pred_hbm, rel_hbm, out_hbm, pred_buf, rel_buf, st_pv,
                  st_pi, st_tp, st_rv, st_tr, gath, ndcg_buf, gsem):
    wid = lax.axis_index("s") * NC + lax.axis_index("c")
    lane = lax.iota(jnp.int32, 16)
    w_vec = jnp.zeros((16,), jnp.float32)
    for k in range(K):
      w_vec = jnp.where(lane == k, float(_W[k]), w_vec)
    neg_inf = jnp.full((16,), -jnp.inf, jnp.float32)
    pos_inf = jnp.full((16,), jnp.inf, jnp.float32)

    def new_threshold(nv):
      t = jnp.min(jnp.where(lane < K, nv, pos_inf))
      return jnp.broadcast_to(t, (16,))

    def merge_pred(vj, base_i):
      sv, si = plsc.sort_key_val(vj, lane + base_i, descending=True)
      rsv = lax.rev(sv, (0,))
      rsi = lax.rev(si, (0,))
      cur_v = st_pv[...]
      cur_i = st_pi[...]
      take = rsv > cur_v
      nv = jnp.where(take, rsv, cur_v)
      ni = jnp.where(take, rsi, cur_i)
      nv, ni = plsc.sort_key_val(nv, ni, descending=True)
      st_pv[...] = nv
      st_pi[...] = ni
      st_tp[...] = new_threshold(nv)

    def merge_rel(vj):
      sv, _ = plsc.sort_key_val(vj, vj, descending=True)
      rsv = lax.rev(sv, (0,))
      cur = st_rv[...]
      take = rsv > cur
      nv = jnp.where(take, rsv, cur)
      nv, _ = plsc.sort_key_val(nv, nv, descending=True)
      st_rv[...] = nv
      st_tr[...] = new_threshold(nv)

    def row_body(rl, carry):
      row = wid * ROWS_PW + rl
      rbase = row * N
      st_pv[...] = neg_inf
      st_pi[...] = jnp.zeros((16,), jnp.int32)
      st_tp[...] = neg_inf
      st_rv[...] = neg_inf
      st_tr[...] = neg_inf

      def chunk_body(ci, c2):
        off = rbase + ci * CHUNK
        pltpu.sync_copy(pred_hbm.at[pl.ds(off, CHUNK)], pred_buf)
        pltpu.sync_copy(rel_hbm.at[pl.ds(off, CHUNK)], rel_buf)
        cbase = ci * CHUNK

        def group_body(g, c3):
          gb = pl.multiple_of(g * GROUP, GROUP)
          pv = [pred_buf[pl.ds(gb + 16 * j, 16)] for j in range(NVREG)]
          pm = pv[0]
          for v in pv[1:]:
            pm = jnp.maximum(pm, v)

          @pl.when(jnp.any(pm > st_tp[...]))
          def _():
            for j in range(NVREG):
              vj = pv[j]

              def do_merge(vj=vj, j=j):
                merge_pred(vj, cbase + gb + 16 * j)

              pl.when(jnp.any(vj > st_tp[...]))(do_merge)

          rv = [rel_buf[pl.ds(gb + 16 * j, 16)] for j in range(NVREG)]
          rm = rv[0]
          for v in rv[1:]:
            rm = jnp.maximum(rm, v)

          @pl.when(jnp.any(rm > st_tr[...]))
          def _():
            for j in range(NVREG):
              vj = rv[j]

              def do_merge(vj=vj):
                merge_rel(vj)

              pl.when(jnp.any(vj > st_tr[...]))(do_merge)

          return c3

        return lax.fori_loop(0, NGROUPS, group_body, c2)

      lax.fori_loop(0, NCHUNKS, chunk_body, 0)

      flat_idx = st_pi[...] + rbase
      pltpu.async_copy(rel_hbm.at[flat_idx], gath, gsem).wait()
      g = gath[...]
      dcg = jnp.sum(jnp.where(lane < K, g * w_vec, 0.0))
      idcg = jnp.sum(jnp.where(lane < K, st_rv[...] * w_vec, 0.0))
      ndcg_v = jnp.broadcast_to(dcg, (16,)) / (
          jnp.broadcast_to(idcg, (16,)) + 1e-8)
      plsc.store_scatter(
          ndcg_buf, [jnp.broadcast_to(rl, (16,)).astype(jnp.int32)], ndcg_v,
          mask=lane == 0)
      return carry

    lax.fori_loop(0, ROWS_PW, row_body, 0)
    pltpu.sync_copy(ndcg_buf, out_hbm.at[pl.ds(wid * ROWS_PW, ROWS_PW)])

  return ndcg_kernel(pred_flat, rel_flat)


def kernel(predictions, relevance_scores):
  pred_flat = predictions.reshape(-1)
  rel_flat = relevance_scores.reshape(-1)
  ndcg = _ndcg_rows(pred_flat, rel_flat)
  return 1.0 - jnp.mean(ndcg)


# speedup vs baseline: 1.9976x; 1.4308x over previous
import functools

import numpy as np
import jax
import jax.numpy as jnp
from jax import lax
from jax.experimental import pallas as pl
from jax.experimental.pallas import tpu as pltpu
from jax.experimental.pallas import tpu_sc as plsc

B = 1024
N = 100000
K = 10
NC = 2
NS = 16
NW = NC * NS
ROWS_PW = B // NW
CHUNK = 20000
GROUP = 400
NVREG = GROUP // 16
NGROUPS = CHUNK // GROUP
NCHUNKS = N // CHUNK
SURV = GROUP + 16

_W = np.zeros(16, np.float32)
_W[:K] = (1.0 / np.log2(np.arange(1, K + 1, dtype=np.float64) + 1.0)).astype(
    np.float32)


def _ndcg_rows(pred_flat, rel_flat):
  mesh = plsc.VectorSubcoreMesh(
      core_axis_name="c", subcore_axis_name="s", num_cores=NC,
      num_subcores=NS)

  @functools.partial(
      pl.kernel,
      out_type=jax.ShapeDtypeStruct((B,), jnp.float32),
      mesh=mesh,
      compiler_params=pltpu.CompilerParams(needs_layout_passes=False),
      scratch_types=[
          pltpu.VMEM((2 * CHUNK,), jnp.float32),
          pltpu.VMEM((2 * CHUNK,), jnp.float32),
          pltpu.VMEM((16,), jnp.float32),
          pltpu.VMEM((16,), jnp.int32),
          pltpu.VMEM((16,), jnp.float32),
          pltpu.VMEM((16,), jnp.float32),
          pltpu.VMEM((16,), jnp.float32),
          pltpu.VMEM((SURV,), jnp.float32),
          pltpu.VMEM((SURV,), jnp.int32),
          pltpu.VMEM((16,), jnp.float32),
          pltpu.VMEM((ROWS_PW,), jnp.float32),
          pltpu.SemaphoreType.DMA,
          pltpu.SemaphoreType.DMA,
          pltpu.SemaphoreType.DMA,
      ],
  )
  def ndcg_kernel(pred_hbm, rel_hbm, out_hbm, pred_buf, rel_buf, st_pv,
                  st_pi, st_tp, st_rv, st_tr, sv_buf, si_buf, gath, ndcg_buf,
                  psem, rsem, gsem):
    wid = lax.axis_index("s") * NC + lax.axis_index("c")
    lane = lax.iota(jnp.int32, 16)
    w_vec = jnp.zeros((16,), jnp.float32)
    for k in range(K):
      w_vec = jnp.where(lane == k, float(_W[k]), w_vec)
    neg_inf = jnp.full((16,), -jnp.inf, jnp.float32)
    pos_inf = jnp.full((16,), jnp.inf, jnp.float32)

    def bslice(buf, parity):
      return buf.at[pl.ds(pl.multiple_of(parity * CHUNK, CHUNK), CHUNK)]

    def fetch(off, parity):
      pltpu.async_copy(pred_hbm.at[pl.ds(off, CHUNK)],
                       bslice(pred_buf, parity), psem)
      pltpu.async_copy(rel_hbm.at[pl.ds(off, CHUNK)],
                       bslice(rel_buf, parity), rsem)

    def wait_fetch(off, parity):
      pltpu.make_async_copy(pred_hbm.at[pl.ds(off, CHUNK)],
                            bslice(pred_buf, parity), psem).wait()
      pltpu.make_async_copy(rel_hbm.at[pl.ds(off, CHUNK)],
                            bslice(rel_buf, parity), rsem).wait()

    def new_threshold(nv):
      t = jnp.min(jnp.where(lane < K, nv, pos_inf))
      return jnp.broadcast_to(t, (16,))

    def merge_pred(vj, iv):
      sv, si = plsc.sort_key_val(vj, iv, descending=True)
      rsv = lax.rev(sv, (0,))
      rsi = lax.rev(si, (0,))
      cur_v = st_pv[...]
      cur_i = st_pi[...]
      take = rsv > cur_v
      nv = jnp.where(take, rsv, cur_v)
      ni = jnp.where(take, rsi, cur_i)
      nv, ni = plsc.sort_key_val(nv, ni, descending=True)
      st_pv[...] = nv
      st_pi[...] = ni
      st_tp[...] = new_threshold(nv)

    def merge_rel(vj):
      sv, _ = plsc.sort_key_val(vj, vj, descending=True)
      rsv = lax.rev(sv, (0,))
      cur = st_rv[...]
      take = rsv > cur
      nv = jnp.where(take, rsv, cur)
      nv, _ = plsc.sort_key_val(nv, nv, descending=True)
      st_rv[...] = nv
      st_tr[...] = new_threshold(nv)

    first_off = wid * ROWS_PW * N
    fetch(first_off, 0)

    def row_body(rl, carry):
      row = wid * ROWS_PW + rl
      rbase = row * N
      st_pv[...] = neg_inf
      st_pi[...] = jnp.zeros((16,), jnp.int32)
      st_tp[...] = neg_inf
      st_rv[...] = neg_inf
      st_tr[...] = neg_inf
      for i in range(SURV // 16):
        sv_buf[pl.ds(16 * i, 16)] = neg_inf

      def chunk_body(ci, c2):
        off = rbase + ci * CHUNK
        parity = (rl + ci) % 2
        wait_fetch(off, parity)
        is_last = jnp.logical_and(rl == ROWS_PW - 1, ci == NCHUNKS - 1)

        @pl.when(jnp.logical_not(is_last))
        def _():
          noff = jnp.where(ci == NCHUNKS - 1, rbase + N, off + CHUNK)
          fetch(noff, 1 - parity)

        pbuf = bslice(pred_buf, parity)
        rbuf = bslice(rel_buf, parity)
        cbase = ci * CHUNK

        def trigger(buf, tvec, gb, col0, with_idx):
          cur = jnp.int32(0)
          for j in range(NVREG):
            v = buf[pl.ds(gb + 16 * j, 16)]
            m = v > tvec
            plsc.store_compressed(sv_buf.at[pl.ds(cur, 16)], v, mask=m)
            if with_idx:
              iv = lane + (col0 + 16 * j)
              plsc.store_compressed(si_buf.at[pl.ds(cur, 16)], iv, mask=m)
            cur = cur + plsc.all_reduce_population_count(m)[0]
          nm = lax.shift_right_logical(cur + 15, 2 + 2)

          def mbody(i, c):
            o = pl.multiple_of(i * 16, 16)
            sv = sv_buf[pl.ds(o, 16)]
            if with_idx:
              si = si_buf[pl.ds(o, 16)]
              merge_pred(sv, si)
            else:
              merge_rel(sv)
            sv_buf[pl.ds(o, 16)] = neg_inf
            return c

          lax.fori_loop(0, nm, mbody, 0)

        def group_body(g, c3):
          gb = pl.multiple_of(g * GROUP, GROUP)
          tp_vec = st_tp[...]
          tr_vec = st_tr[...]
          acc = [None] * 4
          for j in range(NVREG):
            v = pbuf[pl.ds(gb + 16 * j, 16)]
            a = j % 4
            acc[a] = v if acc[a] is None else jnp.maximum(acc[a], v)
          pmax = jnp.maximum(jnp.maximum(acc[0], acc[1]),
                             jnp.maximum(acc[2], acc[3]))
          racc = [None] * 4
          for j in range(NVREG):
            v = rbuf[pl.ds(gb + 16 * j, 16)]
            a = j % 4
            racc[a] = v if racc[a] is None else jnp.maximum(racc[a], v)
          rmax = jnp.maximum(jnp.maximum(racc[0], racc[1]),
                             jnp.maximum(racc[2], racc[3]))
          pcnt = plsc.all_reduce_population_count(pmax > tp_vec)[0]
          rcnt = plsc.all_reduce_population_count(rmax > tr_vec)[0]

          @pl.when(pcnt + rcnt > 0)
          def _():
            @pl.when(pcnt > 0)
            def _():
              trigger(pbuf, tp_vec, gb, cbase + gb, True)

            @pl.when(rcnt > 0)
            def _():
              trigger(rbuf, tr_vec, gb, cbase + gb, False)

          return c3

        return lax.fori_loop(0, NGROUPS, group_body, c2)

      lax.fori_loop(0, NCHUNKS, chunk_body, 0)

      flat_idx = st_pi[...] + rbase
      pltpu.async_copy(rel_hbm.at[flat_idx], gath, gsem).wait()
      g = gath[...]
      dcg = jnp.sum(jnp.where(lane < K, g * w_vec, 0.0))
      idcg = jnp.sum(jnp.where(lane < K, st_rv[...] * w_vec, 0.0))
      ndcg_v = jnp.broadcast_to(dcg, (16,)) / (
          jnp.broadcast_to(idcg, (16,)) + 1e-8)
      plsc.store_scatter(
          ndcg_buf, [jnp.broadcast_to(rl, (16,)).astype(jnp.int32)], ndcg_v,
          mask=lane == 0)
      return carry

    lax.fori_loop(0, ROWS_PW, row_body, 0)
    pltpu.sync_copy(ndcg_buf, out_hbm.at[pl.ds(wid * ROWS_PW, ROWS_PW)])

  return ndcg_kernel(pred_flat, rel_flat)


def kernel(predictions, relevance_scores):
  pred_flat = predictions.reshape(-1)
  rel_flat = relevance_scores.reshape(-1)
  ndcg = _ndcg_rows(pred_flat, rel_flat)
  return 1.0 - jnp.mean(ndcg)


# speedup vs baseline: 2.8188x; 1.4111x over previous
import functools

import numpy as np
import jax
import jax.numpy as jnp
from jax import lax
from jax.experimental import pallas as pl
from jax.experimental.pallas import tpu as pltpu
from jax.experimental.pallas import tpu_sc as plsc

B = 1024
N = 100000
K = 10
NC = 2
NS = 16
NW = NC * NS
ROWS_PW = B // NW
RB = 8
NBLK = ROWS_PW // RB
CHUNK = 3200
NCH = 31
TAIL = N - NCH * CHUNK
GROUP = 400
NVREG = GROUP // 16
NG_REG = CHUNK // GROUP
NG_TAIL = TAIL // GROUP
SURV = GROUP + 16

_W = np.zeros(16, np.float32)
_W[:K] = (1.0 / np.log2(np.arange(1, K + 1, dtype=np.float64) + 1.0)).astype(
    np.float32)


def _ndcg_rows(predictions, relevance):
  mesh = plsc.VectorSubcoreMesh(
      core_axis_name="c", subcore_axis_name="s", num_cores=NC,
      num_subcores=NS)

  @functools.partial(
      pl.kernel,
      out_type=jax.ShapeDtypeStruct((B,), jnp.float32),
      mesh=mesh,
      compiler_params=pltpu.CompilerParams(needs_layout_passes=False),
      scratch_types=[
          pltpu.VMEM((2 * RB, CHUNK), jnp.float32),
          pltpu.VMEM((2 * RB, CHUNK), jnp.float32),
          pltpu.VMEM((RB, TAIL), jnp.float32),
          pltpu.VMEM((RB, TAIL), jnp.float32),
          pltpu.VMEM((RB * 16,), jnp.float32),
          pltpu.VMEM((RB * 16,), jnp.float32),
          pltpu.VMEM((RB * 16,), jnp.float32),
          pltpu.VMEM((RB * 16,), jnp.float32),
          pltpu.VMEM((RB * 16,), jnp.float32),
          pltpu.VMEM((SURV,), jnp.float32),
          pltpu.VMEM((SURV,), jnp.float32),
          pltpu.VMEM((ROWS_PW,), jnp.float32),
          pltpu.SemaphoreType.DMA,
          pltpu.SemaphoreType.DMA,
      ],
  )
  def ndcg_kernel(pred_hbm, rel_hbm, out_hbm, pred_buf, rel_buf, pred_tl,
                  rel_tl, st_pv, st_pr, st_tp, st_rv, st_tr, sv_buf, sp_buf,
                  ndcg_buf, psem, rsem):
    wid = lax.axis_index("s") * NC + lax.axis_index("c")
    lane = lax.iota(jnp.int32, 16)
    w_vec = jnp.zeros((16,), jnp.float32)
    for k in range(K):
      w_vec = jnp.where(lane == k, float(_W[k]), w_vec)
    neg_inf = jnp.full((16,), -jnp.inf, jnp.float32)
    pos_inf = jnp.full((16,), jnp.inf, jnp.float32)
    row0 = wid * ROWS_PW

    def fetch_reg(b, c, parity):
      r8 = row0 + b * RB
      pltpu.async_copy(pred_hbm.at[pl.ds(r8, RB), pl.ds(c * CHUNK, CHUNK)],
                       pred_buf.at[pl.ds(parity * RB, RB)], psem)
      pltpu.async_copy(rel_hbm.at[pl.ds(r8, RB), pl.ds(c * CHUNK, CHUNK)],
                       rel_buf.at[pl.ds(parity * RB, RB)], rsem)

    def wait_reg(b, c, parity):
      r8 = row0 + b * RB
      pltpu.make_async_copy(
          pred_hbm.at[pl.ds(r8, RB), pl.ds(c * CHUNK, CHUNK)],
          pred_buf.at[pl.ds(parity * RB, RB)], psem).wait()
      pltpu.make_async_copy(
          rel_hbm.at[pl.ds(r8, RB), pl.ds(c * CHUNK, CHUNK)],
          rel_buf.at[pl.ds(parity * RB, RB)], rsem).wait()

    def fetch_tail(b):
      r8 = row0 + b * RB
      pltpu.async_copy(pred_hbm.at[pl.ds(r8, RB), pl.ds(NCH * CHUNK, TAIL)],
                       pred_tl, psem)
      pltpu.async_copy(rel_hbm.at[pl.ds(r8, RB), pl.ds(NCH * CHUNK, TAIL)],
                       rel_tl, rsem)

    def wait_tail(b):
      r8 = row0 + b * RB
      pltpu.make_async_copy(
          pred_hbm.at[pl.ds(r8, RB), pl.ds(NCH * CHUNK, TAIL)],
          pred_tl, psem).wait()
      pltpu.make_async_copy(
          rel_hbm.at[pl.ds(r8, RB), pl.ds(NCH * CHUNK, TAIL)],
          rel_tl, rsem).wait()

    def new_threshold(nv):
      t = jnp.min(jnp.where(lane < K, nv, pos_inf))
      return jnp.broadcast_to(t, (16,))

    def merge_pred(so, vj, pj):
      sv, sp = plsc.sort_key_val(vj, pj, descending=True)
      rsv = lax.rev(sv, (0,))
      rsp = lax.rev(sp, (0,))
      cur_v = st_pv[pl.ds(so, 16)]
      cur_p = st_pr[pl.ds(so, 16)]
      take = rsv > cur_v
      nv = jnp.where(take, rsv, cur_v)
      np_ = jnp.where(take, rsp, cur_p)
      nv, np_ = plsc.sort_key_val(nv, np_, descending=True)
      st_pv[pl.ds(so, 16)] = nv
      st_pr[pl.ds(so, 16)] = np_
      st_tp[pl.ds(so, 16)] = new_threshold(nv)

    def merge_rel(so, vj):
      sv, _ = plsc.sort_key_val(vj, vj, descending=True)
      rsv = lax.rev(sv, (0,))
      cur = st_rv[pl.ds(so, 16)]
      take = rsv > cur
      nv = jnp.where(take, rsv, cur)
      nv, _ = plsc.sort_key_val(nv, nv, descending=True)
      st_rv[pl.ds(so, 16)] = nv
      st_tr[pl.ds(so, 16)] = new_threshold(nv)

    def scan_chunk(pbuf, rbuf, prow0, ngroups):
      def row_scan(r, carry):
        so = pl.multiple_of(r * 16, 16)
        prow = prow0 + r

        def trigger_pred(tvec, gb):
          cur = jnp.int32(0)
          for j in range(NVREG):
            v = pbuf[prow, pl.ds(gb + 16 * j, 16)]
            pv = rbuf[prow, pl.ds(gb + 16 * j, 16)]
            m = v > tvec
            plsc.store_compressed(sv_buf.at[pl.ds(cur, 16)], v, mask=m)
            plsc.store_compressed(sp_buf.at[pl.ds(cur, 16)], pv, mask=m)
            cur = cur + plsc.all_reduce_population_count(m)[0]
          nm = lax.shift_right_logical(cur + 15, 4)

          def mbody(i, c):
            o = pl.multiple_of(i * 16, 16)
            merge_pred(so, sv_buf[pl.ds(o, 16)], sp_buf[pl.ds(o, 16)])
            sv_buf[pl.ds(o, 16)] = neg_inf
            return c

          lax.fori_loop(0, nm, mbody, 0)

        def trigger_rel(tvec, gb):
          cur = jnp.int32(0)
          for j in range(NVREG):
            v = rbuf[prow, pl.ds(gb + 16 * j, 16)]
            m = v > tvec
            plsc.store_compressed(sv_buf.at[pl.ds(cur, 16)], v, mask=m)
            cur = cur + plsc.all_reduce_population_count(m)[0]
          nm = lax.shift_right_logical(cur + 15, 4)

          def mbody(i, c):
            o = pl.multiple_of(i * 16, 16)
            merge_rel(so, sv_buf[pl.ds(o, 16)])
            sv_buf[pl.ds(o, 16)] = neg_inf
            return c

          lax.fori_loop(0, nm, mbody, 0)

        def group_body(g, c3):
          gb = pl.multiple_of(g * GROUP, GROUP)
          tp_vec = st_tp[pl.ds(so, 16)]
          tr_vec = st_tr[pl.ds(so, 16)]
          acc = [None] * 4
          for j in range(NVREG):
            v = pbuf[prow, pl.ds(gb + 16 * j, 16)]
            a = j % 4
            acc[a] = v if acc[a] is None else jnp.maximum(acc[a], v)
          pmax = jnp.maximum(jnp.maximum(acc[0], acc[1]),
                             jnp.maximum(acc[2], acc[3]))
          racc = [None] * 4
          for j in range(NVREG):
            v = rbuf[prow, pl.ds(gb + 16 * j, 16)]
            a = j % 4
            racc[a] = v if racc[a] is None else jnp.maximum(racc[a], v)
          rmax = jnp.maximum(jnp.maximum(racc[0], racc[1]),
                             jnp.maximum(racc[2], racc[3]))
          pcnt = plsc.all_reduce_population_count(pmax > tp_vec)[0]
          rcnt = plsc.all_reduce_population_count(rmax > tr_vec)[0]

          @pl.when(pcnt + rcnt > 0)
          def _():
            @pl.when(pcnt > 0)
            def _():
              trigger_pred(tp_vec, gb)

            @pl.when(rcnt > 0)
            def _():
              trigger_rel(tr_vec, gb)

          return c3

        return lax.fori_loop(0, ngroups, group_body, carry)

      lax.fori_loop(0, RB, row_scan, 0)

    for i in range(SURV // 16):
      sv_buf[pl.ds(16 * i, 16)] = neg_inf
    fetch_reg(0, 0, 0)

    def block_body(b, carry):
      for i in range(RB):
        so = pl.multiple_of(i * 16, 16)
        st_pv[pl.ds(so, 16)] = neg_inf
        st_pr[pl.ds(so, 16)] = jnp.zeros((16,), jnp.float32)
        st_tp[pl.ds(so, 16)] = neg_inf
        st_rv[pl.ds(so, 16)] = neg_inf
        st_tr[pl.ds(so, 16)] = neg_inf

      def chunk_body(c, c2):
        parity = c % 2
        wait_reg(b, c, parity)

        @pl.when(c < NCH - 1)
        def _():
          fetch_reg(b, c + 1, 1 - parity)

        @pl.when(c == NCH - 1)
        def _():
          fetch_tail(b)

        scan_chunk(pred_buf, rel_buf, parity * RB, NG_REG)
        return c2

      lax.fori_loop(0, NCH, chunk_body, 0)

      wait_tail(b)

      @pl.when(b < NBLK - 1)
      def _():
        fetch_reg(b + 1, 0, 0)

      scan_chunk(pred_tl, rel_tl, 0, NG_TAIL)

      def finalize(r, c4):
        so = pl.multiple_of(r * 16, 16)
        dcg = jnp.sum(jnp.where(lane < K, st_pr[pl.ds(so, 16)] * w_vec, 0.0))
        idcg = jnp.sum(jnp.where(lane < K, st_rv[pl.ds(so, 16)] * w_vec, 0.0))
        ndcg_v = jnp.broadcast_to(dcg, (16,)) / (
            jnp.broadcast_to(idcg, (16,)) + 1e-8)
        plsc.store_scatter(
            ndcg_buf,
            [jnp.broadcast_to(b * RB + r, (16,)).astype(jnp.int32)], ndcg_v,
            mask=lane == 0)
        return c4

      lax.fori_loop(0, RB, finalize, 0)
      return carry

    lax.fori_loop(0, NBLK, block_body, 0)
    pltpu.sync_copy(ndcg_buf, out_hbm.at[pl.ds(row0, ROWS_PW)])

  return ndcg_kernel(predictions, relevance)


def kernel(predictions, relevance_scores):
  ndcg = _ndcg_rows(predictions, relevance_scores)
  return 1.0 - jnp.mean(ndcg)


# speedup vs baseline: 3.7740x; 1.3389x over previous
import functools

import numpy as np
import jax
import jax.numpy as jnp
from jax import lax
from jax.experimental import pallas as pl
from jax.experimental.pallas import tpu as pltpu
from jax.experimental.pallas import tpu_sc as plsc

B = 1024
N = 100000
K = 10
NC = 2
NS = 16
NW = NC * NS
ROWS_PW = B // NW
RB = 8
NBLK = ROWS_PW // RB
CHUNK = 3200
NCH = 31
TAIL = N - NCH * CHUNK
GROUP = 400
NVREG = GROUP // 16
NG_REG = CHUNK // GROUP
NG_TAIL = TAIL // GROUP
SURV = GROUP + 16

_W = np.zeros(16, np.float32)
_W[:K] = (1.0 / np.log2(np.arange(1, K + 1, dtype=np.float64) + 1.0)).astype(
    np.float32)


def _ndcg_rows(predictions, relevance):
  mesh = plsc.VectorSubcoreMesh(
      core_axis_name="c", subcore_axis_name="s", num_cores=NC,
      num_subcores=NS)

  @functools.partial(
      pl.kernel,
      out_type=jax.ShapeDtypeStruct((B,), jnp.float32),
      mesh=mesh,
      compiler_params=pltpu.CompilerParams(needs_layout_passes=False),
      scratch_types=[
          pltpu.VMEM((2 * RB, CHUNK), jnp.float32),
          pltpu.VMEM((2 * RB, CHUNK), jnp.float32),
          pltpu.VMEM((RB, TAIL), jnp.float32),
          pltpu.VMEM((RB, TAIL), jnp.float32),
          pltpu.VMEM((RB * 16,), jnp.float32),
          pltpu.VMEM((RB * 16,), jnp.float32),
          pltpu.VMEM((RB * 16,), jnp.float32),
          pltpu.VMEM((RB * 16,), jnp.float32),
          pltpu.VMEM((RB * 16,), jnp.float32),
          pltpu.VMEM((SURV,), jnp.float32),
          pltpu.VMEM((SURV,), jnp.float32),
          pltpu.VMEM((ROWS_PW,), jnp.float32),
          pltpu.SemaphoreType.DMA,
          pltpu.SemaphoreType.DMA,
      ],
  )
  def ndcg_kernel(pred_hbm, rel_hbm, out_hbm, pred_buf, rel_buf, pred_tl,
                  rel_tl, st_pv, st_pr, st_tp, st_rv, st_tr, sv_buf, sp_buf,
                  ndcg_buf, psem, rsem):
    wid = lax.axis_index("s") * NC + lax.axis_index("c")
    lane = lax.iota(jnp.int32, 16)
    w_vec = jnp.zeros((16,), jnp.float32)
    for k in range(K):
      w_vec = jnp.where(lane == k, float(_W[k]), w_vec)
    neg_inf = jnp.full((16,), -jnp.inf, jnp.float32)
    pos_inf = jnp.full((16,), jnp.inf, jnp.float32)
    row0 = wid * ROWS_PW

    def fetch_reg(b, c, parity):
      r8 = row0 + b * RB
      pltpu.async_copy(pred_hbm.at[pl.ds(r8, RB), pl.ds(c * CHUNK, CHUNK)],
                       pred_buf.at[pl.ds(parity * RB, RB)], psem)
      pltpu.async_copy(rel_hbm.at[pl.ds(r8, RB), pl.ds(c * CHUNK, CHUNK)],
                       rel_buf.at[pl.ds(parity * RB, RB)], rsem)

    def wait_reg(b, c, parity):
      r8 = row0 + b * RB
      pltpu.make_async_copy(
          pred_hbm.at[pl.ds(r8, RB), pl.ds(c * CHUNK, CHUNK)],
          pred_buf.at[pl.ds(parity * RB, RB)], psem).wait()
      pltpu.make_async_copy(
          rel_hbm.at[pl.ds(r8, RB), pl.ds(c * CHUNK, CHUNK)],
          rel_buf.at[pl.ds(parity * RB, RB)], rsem).wait()

    def fetch_tail(b):
      r8 = row0 + b * RB
      pltpu.async_copy(pred_hbm.at[pl.ds(r8, RB), pl.ds(NCH * CHUNK, TAIL)],
                       pred_tl, psem)
      pltpu.async_copy(rel_hbm.at[pl.ds(r8, RB), pl.ds(NCH * CHUNK, TAIL)],
                       rel_tl, rsem)

    def wait_tail(b):
      r8 = row0 + b * RB
      pltpu.make_async_copy(
          pred_hbm.at[pl.ds(r8, RB), pl.ds(NCH * CHUNK, TAIL)],
          pred_tl, psem).wait()
      pltpu.make_async_copy(
          rel_hbm.at[pl.ds(r8, RB), pl.ds(NCH * CHUNK, TAIL)],
          rel_tl, rsem).wait()

    def new_threshold(nv):
      t = jnp.min(jnp.where(lane < K, nv, pos_inf))
      return jnp.broadcast_to(t, (16,))

    def merge_pred_sorted(so, sv, sp):
      rsv = lax.rev(sv, (0,))
      rsp = lax.rev(sp, (0,))
      cur_v = st_pv[pl.ds(so, 16)]
      cur_p = st_pr[pl.ds(so, 16)]
      take = rsv > cur_v
      nv = jnp.where(take, rsv, cur_v)
      np_ = jnp.where(take, rsp, cur_p)
      nv, np_ = plsc.sort_key_val(nv, np_, descending=True)
      st_pv[pl.ds(so, 16)] = nv
      st_pr[pl.ds(so, 16)] = np_
      st_tp[pl.ds(so, 16)] = jnp.broadcast_to(nv[K - 1], (16,))

    def merge_rel_sorted(so, sv):
      rsv = lax.rev(sv, (0,))
      cur = st_rv[pl.ds(so, 16)]
      take = rsv > cur
      nv = jnp.where(take, rsv, cur)
      nv, _ = plsc.sort_key_val(nv, nv, descending=True)
      st_rv[pl.ds(so, 16)] = nv
      st_tr[pl.ds(so, 16)] = jnp.broadcast_to(nv[K - 1], (16,))

    def scan_chunk(pbuf, rbuf, prow0, ngroups):
      def row_scan(r, carry):
        so = pl.multiple_of(r * 16, 16)
        prow = prow0 + r

        def locate(buf, tvec, gb):
          cnt_a = jnp.zeros((16,), jnp.int32)
          cnt_b = jnp.zeros((16,), jnp.int32)
          for j in range(NVREG):
            v = buf[prow, pl.ds(gb + 16 * j, 16)]
            c = plsc.all_reduce_population_count(v > tvec)
            if j < 16:
              cnt_a = jnp.where(lane == j, c, cnt_a)
            else:
              cnt_b = jnp.where(lane == j - 16, c, cnt_b)
          return cnt_a, cnt_b

        def hot_cond(st):
          ma, mb = st
          return jnp.any(ma > 0) | jnp.any(mb > 0)

        def next_hot(ma, mb):
          anyA = jnp.any(ma > 0)
          ja = plsc.all_reduce_ffs(ma > 0)[0]
          jb = plsc.all_reduce_ffs(mb > 0)[0]
          j = jnp.where(anyA, ja, jb + 16)
          ma2 = jnp.where(lane == j, 0, ma)
          mb2 = jnp.where(lane == j - 16, 0, mb)
          return j, ma2, mb2

        def trigger_pred(tvec, gb):
          counts = locate(pbuf, tvec, gb)

          def body(st):
            ma, mb = st
            j, ma2, mb2 = next_hot(ma, mb)
            v = pbuf[prow, pl.ds(gb + 16 * j, 16)]
            pv = rbuf[prow, pl.ds(gb + 16 * j, 16)]
            m = v > st_tp[pl.ds(so, 16)]
            cnt = plsc.all_reduce_population_count(m)[0]
            plsc.store_compressed(sv_buf.at[pl.ds(0, 16)], v, mask=m)
            plsc.store_compressed(sp_buf.at[pl.ds(0, 16)], pv, mask=m)
            sv = sv_buf[pl.ds(0, 16)]
            sp = sp_buf[pl.ds(0, 16)]
            sv_buf[pl.ds(0, 16)] = neg_inf
            sv, sp = lax.cond(
                cnt > 1,
                lambda: tuple(plsc.sort_key_val(sv, sp, descending=True)),
                lambda: (sv, sp))

            @pl.when(cnt > 0)
            def _():
              merge_pred_sorted(so, sv, sp)

            return ma2, mb2

          lax.while_loop(hot_cond, body, counts)

        def trigger_rel(tvec, gb):
          counts = locate(rbuf, tvec, gb)

          def body(st):
            ma, mb = st
            j, ma2, mb2 = next_hot(ma, mb)
            v = rbuf[prow, pl.ds(gb + 16 * j, 16)]
            m = v > st_tr[pl.ds(so, 16)]
            cnt = plsc.all_reduce_population_count(m)[0]
            plsc.store_compressed(sv_buf.at[pl.ds(0, 16)], v, mask=m)
            sv = sv_buf[pl.ds(0, 16)]
            sv_buf[pl.ds(0, 16)] = neg_inf
            sv = lax.cond(
                cnt > 1,
                lambda: plsc.sort_key_val(sv, sv, descending=True)[0],
                lambda: sv)

            @pl.when(cnt > 0)
            def _():
              merge_rel_sorted(so, sv)

            return ma2, mb2

          lax.while_loop(hot_cond, body, counts)

        def group_body(g, c3):
          gb = pl.multiple_of(g * GROUP, GROUP)
          tp_vec = st_tp[pl.ds(so, 16)]
          tr_vec = st_tr[pl.ds(so, 16)]
          acc = [None] * 4
          for j in range(NVREG):
            v = pbuf[prow, pl.ds(gb + 16 * j, 16)]
            a = j % 4
            acc[a] = v if acc[a] is None else jnp.maximum(acc[a], v)
          pmax = jnp.maximum(jnp.maximum(acc[0], acc[1]),
                             jnp.maximum(acc[2], acc[3]))
          racc = [None] * 4
          for j in range(NVREG):
            v = rbuf[prow, pl.ds(gb + 16 * j, 16)]
            a = j % 4
            racc[a] = v if racc[a] is None else jnp.maximum(racc[a], v)
          rmax = jnp.maximum(jnp.maximum(racc[0], racc[1]),
                             jnp.maximum(racc[2], racc[3]))
          pcnt = plsc.all_reduce_population_count(pmax > tp_vec)[0]
          rcnt = plsc.all_reduce_population_count(rmax > tr_vec)[0]

          @pl.when(pcnt + rcnt > 0)
          def _():
            @pl.when(pcnt > 0)
            def _():
              trigger_pred(tp_vec, gb)

            @pl.when(rcnt > 0)
            def _():
              trigger_rel(tr_vec, gb)

          return c3

        return lax.fori_loop(0, ngroups, group_body, carry)

      lax.fori_loop(0, RB, row_scan, 0)

    for i in range(SURV // 16):
      sv_buf[pl.ds(16 * i, 16)] = neg_inf
    fetch_reg(0, 0, 0)

    def block_body(b, carry):
      for i in range(RB):
        so = pl.multiple_of(i * 16, 16)
        st_pv[pl.ds(so, 16)] = neg_inf
        st_pr[pl.ds(so, 16)] = jnp.zeros((16,), jnp.float32)
        st_tp[pl.ds(so, 16)] = neg_inf
        st_rv[pl.ds(so, 16)] = neg_inf
        st_tr[pl.ds(so, 16)] = neg_inf

      def chunk_body(c, c2):
        parity = c % 2
        wait_reg(b, c, parity)

        @pl.when(c < NCH - 1)
        def _():
          fetch_reg(b, c + 1, 1 - parity)

        @pl.when(c == NCH - 1)
        def _():
          fetch_tail(b)

        scan_chunk(pred_buf, rel_buf, parity * RB, NG_REG)
        return c2

      lax.fori_loop(0, NCH, chunk_body, 0)

      wait_tail(b)

      @pl.when(b < NBLK - 1)
      def _():
        fetch_reg(b + 1, 0, 0)

      scan_chunk(pred_tl, rel_tl, 0, NG_TAIL)

      def finalize(r, c4):
        so = pl.multiple_of(r * 16, 16)
        dcg = jnp.sum(jnp.where(lane < K, st_pr[pl.ds(so, 16)] * w_vec, 0.0))
        idcg = jnp.sum(jnp.where(lane < K, st_rv[pl.ds(so, 16)] * w_vec, 0.0))
        ndcg_v = jnp.broadcast_to(dcg, (16,)) / (
            jnp.broadcast_to(idcg, (16,)) + 1e-8)
        plsc.store_scatter(
            ndcg_buf,
            [jnp.broadcast_to(b * RB + r, (16,)).astype(jnp.int32)], ndcg_v,
            mask=lane == 0)
        return c4

      lax.fori_loop(0, RB, finalize, 0)
      return carry

    lax.fori_loop(0, NBLK, block_body, 0)
    pltpu.sync_copy(ndcg_buf, out_hbm.at[pl.ds(row0, ROWS_PW)])

  return ndcg_kernel(predictions, relevance)


def kernel(predictions, relevance_scores):
  ndcg = _ndcg_rows(predictions, relevance_scores)
  return 1.0 - jnp.mean(ndcg)
